# Initial kernel scaffold; baseline (speedup 1.0000x reference)
import jax
import jax.numpy as jnp
from jax.experimental import pallas as pl


def kernel(x, edge_index, W1, b1, W2, b2):
    raise NotImplementedError("write your pallas kernel here")


# speedup vs baseline: 6.4550x; 6.4550x over previous
import functools

import jax
import jax.numpy as jnp
from jax import lax
from jax.experimental import pallas as pl
from jax.experimental.pallas import tpu as pltpu
from jax.experimental.pallas import tpu_sc as plsc

N = 10000
D_IN = 128
D_HID = 256
D_OUT = 128
K = 10
ALPHA = 0.1

NTILES = 32
NSUB = 16
NP = 10240
RPT = NP // NSUB
HRPT = RPT // 2
GROW = 10368

E = 320000
ET = E + N
CHUNK = 128
NCH = 81
EP = NTILES * NCH * CHUNK

NB = 400


WBC = 32


def _step_body(src_hbm, dst_hbm, s_in, c0_in, d2_hbm, s_out,
               src_v, dst_v, rows_v, wb_v, d2_v, agg_sh, sem):
    c = lax.axis_index("c")
    s = lax.axis_index("s")

    pltpu.sync_copy(c0_in.at[pl.ds(c * NP + s * RPT, RPT)],
                    agg_sh.at[pl.ds(s * RPT, RPT)])
    plsc.subcore_barrier()

    for half_id in range(2):
        slice_id = s + half_id * NSUB
        pltpu.sync_copy(src_hbm.at[c * NTILES + slice_id], src_v)
        pltpu.sync_copy(dst_hbm.at[slice_id], dst_v)

        def body_j(j, _):
            pltpu.async_copy(s_in.at[src_v.at[j]], rows_v, sem).wait()
            pltpu.sync_copy(rows_v, agg_sh.at[dst_v.at[j]], add=True)
            return 0

        lax.fori_loop(0, NCH, body_j, 0)
    plsc.subcore_barrier()

    def body_wb(rep, _):
        base = s * RPT + rep * WBC
        pltpu.sync_copy(agg_sh.at[pl.ds(base, WBC)], wb_v)
        pltpu.sync_copy(d2_hbm.at[pl.ds(base, WBC)], d2_v)

        def body_r(r, _):
            dv = d2_v[r]
            for dc in range(128 // 16):
                sl = pl.ds(dc * 16, 16)
                wb_v[r, sl] = wb_v[r, sl] * dv
            return 0

        lax.fori_loop(0, WBC, body_r, 0)
        pltpu.sync_copy(wb_v, s_out.at[pl.ds(c * NP + base, WBC)])
        return 0

    lax.fori_loop(0, RPT // WBC, body_wb, 0)


_step = functools.partial(
    pl.kernel,
    mesh=plsc.VectorSubcoreMesh(core_axis_name="c", subcore_axis_name="s"),
    out_type=jax.ShapeDtypeStruct((2 * NP, 128), jnp.float32),
    scratch_types=[
        pltpu.VMEM((NCH, CHUNK), jnp.int32),
        pltpu.VMEM((NCH, CHUNK), jnp.int32),
        pltpu.VMEM((CHUNK, 128), jnp.float32),
        pltpu.VMEM((WBC, 128), jnp.float32),
        pltpu.VMEM((WBC, 16), jnp.float32),
        pltpu.VMEM_SHARED((GROW, 128), jnp.float32),
        pltpu.SemaphoreType.DMA,
    ],
)(_step_body)


def _tc1_body(x_ref, w_ref, b_ref, deg_ref, s0_ref, c0_ref, d2_ref):
    h = jnp.dot(x_ref[...], w_ref[...],
                preferred_element_type=jnp.float32) + b_ref[...]
    deg = deg_ref[...][:, 0:1]
    sq = jnp.sqrt(deg)
    s0 = h / sq
    c0 = (ALPHA / (1.0 - ALPHA)) * h * sq
    s0_ref[0] = s0[:, :128]
    s0_ref[1] = s0[:, 128:]
    c0_ref[0] = c0[:, :128]
    c0_ref[1] = c0[:, 128:]
    d2_ref[...] = (1.0 - ALPHA) / jnp.maximum(deg_ref[...], 1.0)


def _tc1(x, W1, b1, deg):
    return pl.pallas_call(
        _tc1_body,
        grid=(N // NB,),
        in_specs=[
            pl.BlockSpec((NB, D_IN), lambda i: (i, 0)),
            pl.BlockSpec((D_IN, D_HID), lambda i: (0, 0)),
            pl.BlockSpec((1, D_HID), lambda i: (0, 0)),
            pl.BlockSpec((NB, 16), lambda i: (i, 0)),
        ],
        out_specs=[
            pl.BlockSpec((2, NB, 128), lambda i: (0, i, 0)),
            pl.BlockSpec((2, NB, 128), lambda i: (0, i, 0)),
            pl.BlockSpec((NB, 16), lambda i: (i, 0)),
        ],
        out_shape=[
            jax.ShapeDtypeStruct((2, N, 128), jnp.float32),
            jax.ShapeDtypeStruct((2, N, 128), jnp.float32),
            jax.ShapeDtypeStruct((N, 16), jnp.float32),
        ],
    )(x, W1, b1, deg)


def _tc2_body(s_ref, w_ref, b_ref, deg_ref, s0_ref, c0_ref):
    deg = deg_ref[...][:, 0:1]
    sq = jnp.sqrt(deg)
    h = jnp.concatenate([s_ref[0], s_ref[1]], axis=1) * sq
    x2 = jnp.maximum(h, 0.0)
    h0 = jnp.dot(x2, w_ref[...], preferred_element_type=jnp.float32) + b_ref[...]
    s0 = h0 / sq
    c0 = (ALPHA / (1.0 - ALPHA)) * h0 * sq
    z = jnp.zeros((NB, 64), jnp.float32)
    s0_ref[0] = jnp.concatenate([s0[:, :64], z], axis=1)
    s0_ref[1] = jnp.concatenate([s0[:, 64:], z], axis=1)
    c0_ref[0] = jnp.concatenate([c0[:, :64], z], axis=1)
    c0_ref[1] = jnp.concatenate([c0[:, 64:], z], axis=1)


def _tc2(sK, W2, b2, deg):
    return pl.pallas_call(
        _tc2_body,
        grid=(N // NB,),
        in_specs=[
            pl.BlockSpec((2, NB, 128), lambda i: (0, i, 0)),
            pl.BlockSpec((D_HID, D_OUT), lambda i: (0, 0)),
            pl.BlockSpec((1, D_OUT), lambda i: (0, 0)),
            pl.BlockSpec((NB, 16), lambda i: (i, 0)),
        ],
        out_specs=[
            pl.BlockSpec((2, NB, 128), lambda i: (0, i, 0)),
            pl.BlockSpec((2, NB, 128), lambda i: (0, i, 0)),
        ],
        out_shape=[
            jax.ShapeDtypeStruct((2, N, 128), jnp.float32),
            jax.ShapeDtypeStruct((2, N, 128), jnp.float32),
        ],
    )(sK, W2, b2, deg)


def _tc3_body(s_ref, deg_ref, out_ref):
    sq = jnp.sqrt(deg_ref[...][:, 0:1])
    out_ref[...] = jnp.concatenate(
        [s_ref[0][:, :64], s_ref[1][:, :64]], axis=1) * sq


def _tc3(sK, deg):
    return pl.pallas_call(
        _tc3_body,
        grid=(N // NB,),
        in_specs=[
            pl.BlockSpec((2, NB, 128), lambda i: (0, i, 0)),
            pl.BlockSpec((NB, 16), lambda i: (i, 0)),
        ],
        out_specs=pl.BlockSpec((NB, D_OUT), lambda i: (i, 0)),
        out_shape=jax.ShapeDtypeStruct((N, D_OUT), jnp.float32),
    )(sK, deg)


def _pad_state(a):
    a = jnp.pad(a, ((0, 0), (0, NP - N), (0, 0)))
    return a.reshape(2 * NP, 128)


def kernel(x, edge_index, W1, b1, W2, b2):
    loop = jnp.arange(N, dtype=jnp.int32)
    src = jnp.concatenate([edge_index[0], loop,
                           jnp.zeros((EP - ET,), jnp.int32)])
    dst = jnp.concatenate([edge_index[1], loop,
                           jnp.full((EP - ET,), NP, jnp.int32)])
    src2 = jnp.stack([src, src + NP]).reshape(2 * NTILES, NCH, CHUNK)
    dst_t = dst.reshape(NTILES, NCH, CHUNK)

    ones_s = jnp.ones((2 * NP, 128), jnp.float32)
    zeros_c0 = jnp.zeros((2 * NP, 128), jnp.float32)
    ones_d2 = jnp.ones((NP, 16), jnp.float32)
    deg_p = _step(src2, dst_t, ones_s, zeros_c0, ones_d2)
    deg = deg_p[:N, :16]

    s0, c0, d2 = _tc1(x, W1, b1.reshape(1, D_HID), deg)
    d2_p = jnp.concatenate([d2, jnp.ones((NP - N, 16), jnp.float32)])
    s = _pad_state(s0)
    c0_p = _pad_state(c0)
    for _ in range(K):
        s = _step(src2, dst_t, s, c0_p, d2_p)

    sK = s.reshape(2, NP, 128)[:, :N]
    s0b, c0b = _tc2(sK, W2, b2.reshape(1, D_OUT), deg)
    s = _pad_state(s0b)
    c0_p2 = _pad_state(c0b)
    for _ in range(K):
        s = _step(src2, dst_t, s, c0_p2, d2_p)

    return _tc3(s.reshape(2, NP, 128)[:, :N], deg)
